# baseline (speedup 1.0000x reference)
import functools

import jax
import jax.numpy as jnp
from jax import lax
from jax.experimental import pallas as pl
from jax.experimental.pallas import tpu as pltpu
from jax.experimental.pallas import tpu_sc as plsc

SLOPE = 0.2
E_TILE = 2560
N_TILE = 1000

N_NODES = 10000
N_EDGES = 320000
NS = 16
E_CHUNKS = (81920, 238080)
E_OFFS = (0, 81920)
BS = 80
SBLK = 64
NW = 32
G = 40
NPT = 624
NREM = N_NODES - NS * NPT
NPT = 624
NREM = N_NODES - NS * NPT


def _leaky(x):
    return jnp.where(x >= 0, x, SLOPE * x)


def _edge_mlp_body(xt_ref, ea_ref, w1a_ref, w1b_ref, b1_ref, w2_ref, b2_ref,
                   mma_ref, mmb_ref):
    h = xt_ref[...] @ w1a_ref[...] + ea_ref[...] @ w1b_ref[...] + b1_ref[...]
    h = _leaky(h)
    m = h @ w2_ref[...] + b2_ref[...]
    m2 = m * m
    m3 = m2 * m
    m4 = m2 * m2
    mma_ref[0] = m[:, :128]
    mma_ref[1] = m2[:, :128]
    mma_ref[2] = m3[:, :128]
    mma_ref[3] = m4[:, :128]
    mmb_ref[0] = m[:, 128:]
    mmb_ref[1] = m2[:, 128:]
    mmb_ref[2] = m3[:, 128:]
    mmb_ref[3] = m4[:, 128:]


def _edge_mlp(xt_g, ea_full, p, W1a, W1b, b1, W2, b2):
    e = xt_g.shape[0]
    grid = e // E_TILE
    off = E_OFFS[p] // E_TILE
    row_spec = pl.BlockSpec((E_TILE, 128), lambda i: (i, 0))
    ea_spec = pl.BlockSpec((E_TILE, 128), lambda i: (i + off, 0))
    full = lambda shape: pl.BlockSpec(shape, lambda i: (0,) * len(shape))
    out_sd = jax.ShapeDtypeStruct((4, e, 128), jnp.float32)
    return pl.pallas_call(
        _edge_mlp_body,
        grid=(grid,),
        in_specs=[row_spec, ea_spec,
                  full((128, 256)), full((128, 256)), full((1, 256)),
                  full((256, 256)), full((1, 256))],
        out_specs=[pl.BlockSpec((4, E_TILE, 128), lambda i: (0, i, 0))] * 2,
        out_shape=[out_sd] * 2,
    )(xt_g, ea_full, W1a, W1b, b1, W2, b2)


def _zero_slice(src_zeros, dst, s):
    row0 = pl.multiple_of(s * NPT, 8)
    pltpu.sync_copy(src_zeros.at[pl.ds(row0, NPT)], dst.at[pl.ds(row0, NPT)])

    @pl.when(s == NS - 1)
    def _():
        pltpu.sync_copy(src_zeros.at[pl.ds(NS * NPT, NREM)],
                        dst.at[pl.ds(NS * NPT, NREM)])


def _flush_slice(src_acc, dst, s):
    row0 = pl.multiple_of(s * NPT, 8)
    pltpu.sync_copy(src_acc.at[pl.ds(row0, NPT)], dst.at[pl.ds(row0, NPT)])

    @pl.when(s == NS - 1)
    def _():
        pltpu.sync_copy(src_acc.at[pl.ds(NS * NPT, NREM)],
                        dst.at[pl.ds(NS * NPT, NREM)])


def _make_sc_body(ept, blocks):
    def _sc_body(mma_ref, mmb_ref, src_ref, zer_ref,
                 out_a_ref, out_b_ref, outc_ref,
                 src_v, buf0, buf1, buf2, acc, g0, g1, g2, ss):
        c = lax.axis_index("c")
        s = lax.axis_index("s")
        e_base = pl.multiple_of(s * ept, 8)
        bufs = (buf0, buf1, buf2)
        gsems = (g0, g1, g2)

        def acc_at(t):
            return acc.at[src_v.at[t]]

        def stage_src(blk):
            pltpu.sync_copy(src_ref.at[s, pl.ds(blk * SBLK, SBLK)], src_v)

        def scatter_block(mm_ref, m, blk, live):
            t_base = blk * SBLK
            ntri = live // 3
            tail = live % 3

            def triple(i, carry):
                r0 = i * 3
                ds_ = [
                    pltpu.async_copy(
                        mm_ref.at[m,
                                  pl.ds(e_base + (t_base + r0 + k) * BS, BS)],
                        bufs[k], gsems[k])
                    for k in range(3)
                ]
                ss_ = []
                for k in range(3):
                    ds_[k].wait()
                    ss_.append(pltpu.async_copy(bufs[k], acc_at(r0 + k), ss,
                                                add=True))
                for d in ss_:
                    d.wait()
                return carry

            lax.fori_loop(0, ntri, triple, 0)
            for k in range(tail):
                r = ntri * 3 + k
                pltpu.sync_copy(
                    mm_ref.at[m, pl.ds(e_base + (t_base + r) * BS, BS)], buf0)
                st = pltpu.async_copy(buf0, acc_at(r), ss, add=True)
                st.wait()

        for j in range(2):
            m = c * 2 + j
            for half in range(2):
                mm_ref = mma_ref if half == 0 else mmb_ref
                out_ref = out_a_ref if half == 0 else out_b_ref
                _zero_slice(zer_ref, acc, s)
                plsc.subcore_barrier()
                for blk, live in enumerate(blocks):
                    stage_src(blk)
                    scatter_block(mm_ref, m, blk, live)
                plsc.subcore_barrier()
                _flush_slice(acc, out_ref.at[m], s)

        def fill(r, carry):
            for q in range(8):
                buf1[r, pl.ds(q * 16, 16)] = jnp.ones((16,), jnp.float32)
            return carry

        lax.fori_loop(0, BS, fill, 0)
        _zero_slice(zer_ref, acc, s)
        plsc.subcore_barrier()

        def count_block(nch):
            def cbody(i, carry):
                t0 = i * 2
                s0 = pltpu.async_copy(buf1, acc_at(t0), ss, add=True)
                s1 = pltpu.async_copy(buf1, acc_at(t0 + 1), ss, add=True)
                s0.wait()
                s1.wait()
                return carry

            lax.fori_loop(0, nch // 2, cbody, 0)
            if nch % 2:
                st = pltpu.async_copy(buf1, acc_at(nch - 1), ss, add=True)
                st.wait()

        for blk, live in enumerate(blocks):
            @pl.when(c == blk % 2)
            def _():
                stage_src(blk)
                count_block(live)

        plsc.subcore_barrier()

        @pl.when(c == 0)
        def _():
            _flush_slice(acc, outc_ref.at[0], s)

        @pl.when(c == 1)
        def _():
            _flush_slice(acc, outc_ref.at[1], s)

    return _sc_body


def _sc_scatter(mma, mmb, src4, zeros):
    e = mma.shape[1]
    ept = e // NS
    nch = ept // BS
    nblk = (nch + SBLK - 1) // SBLK
    blocks = [min(SBLK, nch - b * SBLK) for b in range(nblk)]
    f = pl.kernel(
        _make_sc_body(ept, blocks),
        out_type=[
            jax.ShapeDtypeStruct((4, N_NODES, 128), jnp.float32),
            jax.ShapeDtypeStruct((4, N_NODES, 128), jnp.float32),
            jax.ShapeDtypeStruct((2, N_NODES, 128), jnp.float32),
        ],
        mesh=plsc.VectorSubcoreMesh(core_axis_name="c", subcore_axis_name="s"),
        scratch_types=[
            pltpu.VMEM((SBLK, BS), jnp.int32),
            pltpu.VMEM((BS, 128), jnp.float32),
            pltpu.VMEM((BS, 128), jnp.float32),
            pltpu.VMEM((BS, 128), jnp.float32),
            pltpu.VMEM_SHARED((N_NODES, 128), jnp.float32),
            pltpu.SemaphoreType.DMA,
            pltpu.SemaphoreType.DMA,
            pltpu.SemaphoreType.DMA,
            pltpu.SemaphoreType.DMA,
        ],
    )
    return f(mma, mmb, src4, zeros)


def _make_gather_body(rpt, ncg):
    def _gather_body(xt_ref, idx_ref, out_ref, idx_v, gb0, gb1, gb2, gb3,
                     s0, s1, s2, s3, os):
        c = lax.axis_index("c")
        s = lax.axis_index("s")
        w = s * 2 + c
        base = pl.multiple_of(w * rpt, 8)
        pltpu.sync_copy(idx_ref.at[w], idx_v)
        gbs = (gb0, gb1, gb2, gb3)
        gsems = (s0, s1, s2, s3)

        def quad(i, carry):
            t0 = i * 4
            ds_ = [
                pltpu.async_copy(xt_ref.at[idx_v.at[t0 + k]], gbs[k],
                                 gsems[k])
                for k in range(4)
            ]
            os_ = []
            for k in range(4):
                ds_[k].wait()
                os_.append(pltpu.async_copy(
                    gbs[k], out_ref.at[pl.ds(base + (t0 + k) * G, G)], os))
            for d in os_:
                d.wait()
            return carry

        lax.fori_loop(0, ncg // 4, quad, 0)
        for k in range(ncg % 4):
            t = (ncg // 4) * 4 + k
            d = pltpu.async_copy(xt_ref.at[idx_v.at[t]], gb0, s0)
            d.wait()
            pltpu.sync_copy(gb0, out_ref.at[pl.ds(base + t * G, G)])

    return _gather_body


def _sc_gather(x_t, idx3):
    nw, ncg, _ = idx3.shape
    rpt = ncg * G
    f = pl.kernel(
        _make_gather_body(rpt, ncg),
        out_type=jax.ShapeDtypeStruct((nw * rpt, 128), jnp.float32),
        mesh=plsc.VectorSubcoreMesh(core_axis_name="c", subcore_axis_name="s"),
        scratch_types=[
            pltpu.VMEM((ncg, G), jnp.int32),
            pltpu.VMEM((G, 128), jnp.float32),
            pltpu.VMEM((G, 128), jnp.float32),
            pltpu.VMEM((G, 128), jnp.float32),
            pltpu.VMEM((G, 128), jnp.float32),
            pltpu.SemaphoreType.DMA,
            pltpu.SemaphoreType.DMA,
            pltpu.SemaphoreType.DMA,
            pltpu.SemaphoreType.DMA,
            pltpu.SemaphoreType.DMA,
        ],
    )
    return f(x_t, idx3)


def _node_body(oma0_ref, oma1_ref, omb0_ref, omb1_ref, rec_ref, xs_ref,
               xu_ref, u1_ref, c1_ref, u2_ref, c2_ref, h_ref):
    r = rec_ref[:, 0:1]

    def stats(om):
        mu1 = om[0] * r
        mu2 = om[1] * r
        mu3 = om[2] * r
        mu4 = om[3] * r
        var = _leaky(mu2 - mu1 * mu1)
        std = jnp.sqrt(var + 1e-6)
        cen3 = mu3 - 3.0 * mu1 * mu2 + 2.0 * mu1 * mu1 * mu1
        cen4 = (mu4 - 4.0 * mu1 * mu3 + 6.0 * mu1 * mu1 * mu2
                - 3.0 * mu1 * mu1 * mu1 * mu1)
        s3 = std * std * std
        return mu1, std, cen3 / s3, cen4 / (s3 * std)

    mu1a, stda, skewa, kurta = stats(oma0_ref[...] + oma1_ref[...])
    mu1b, stdb, skewb, kurtb = stats(omb0_ref[...] + omb1_ref[...])
    xu = jnp.broadcast_to(xu_ref[...], (N_TILE, 128))
    hin = jnp.concatenate([xs_ref[...], mu1a, mu1b, stda, stdb,
                           skewa, skewb, kurta, kurtb, xu], axis=1)
    z = _leaky(hin @ u1_ref[...] + c1_ref[...])
    h_ref[...] = z @ u2_ref[...] + c2_ref[...]


def _node_mlp(oma0, oma1, omb0, omb1, rec128, x_s, x_u, U1, c1, U2, c2):
    n = x_s.shape[0]
    grid = n // N_TILE
    full = lambda shape: pl.BlockSpec(shape, lambda i: (0,) * len(shape))
    om_spec = pl.BlockSpec((4, N_TILE, 128), lambda i: (0, i, 0))
    return pl.pallas_call(
        _node_body,
        grid=(grid,),
        in_specs=[om_spec, om_spec, om_spec, om_spec,
                  pl.BlockSpec((N_TILE, 128), lambda i: (i, 0)),
                  pl.BlockSpec((N_TILE, 128), lambda i: (i, 0)),
                  full((1, 128)),
                  full((1280, 1280)), full((1, 1280)),
                  full((1280, 128)), full((1, 128))],
        out_specs=pl.BlockSpec((N_TILE, 128), lambda i: (i, 0)),
        out_shape=jax.ShapeDtypeStruct((n, 128), jnp.float32),
    )(oma0, oma1, omb0, omb1, rec128, x_s, x_u, U1, c1, U2, c2)


def _bn_body(h_ref, g_ref, b_ref, out_ref):
    h = h_ref[...]
    mu = jnp.mean(h, axis=0, keepdims=True)
    v = jnp.mean((h - mu) ** 2, axis=0, keepdims=True)
    out_ref[...] = g_ref[...] * (h - mu) / jnp.sqrt(v + 1e-5) + b_ref[...]


def _batchnorm(h, gamma, beta):
    n = h.shape[0]
    return pl.pallas_call(
        _bn_body,
        in_specs=[pl.BlockSpec((n, 128), lambda: (0, 0)),
                  pl.BlockSpec((1, 128), lambda: (0, 0)),
                  pl.BlockSpec((1, 128), lambda: (0, 0))],
        out_specs=pl.BlockSpec((n, 128), lambda: (0, 0)),
        out_shape=jax.ShapeDtypeStruct((n, 128), jnp.float32),
    )(h, gamma.reshape(1, 128), beta.reshape(1, 128))


def kernel(x_s, x_t, edge_index, edge_attr, x_u, W1, b1, W2, b2, U1, c1, U2,
           c2, gamma, beta):
    src = edge_index[0]
    tgt = edge_index[1]

    W1a = W1[:128]
    W1b = W1[128:]

    zeros = jnp.zeros((N_NODES, 128), jnp.float32)
    b1r = b1.reshape(1, 256)
    b2r = b2.reshape(1, 256)

    oms = []
    cnt = None
    for p in range(len(E_CHUNKS)):
        ec = E_CHUNKS[p]
        sl = slice(E_OFFS[p], E_OFFS[p] + ec)
        xt_g = _sc_gather(x_t, tgt[sl].reshape(NW, ec // (NW * G), G))
        mma, mmb = _edge_mlp(xt_g, edge_attr, p, W1a, W1b, b1r, W2, b2r)
        nch = ec // (NS * BS)
        nchp = ((nch + SBLK - 1) // SBLK) * SBLK
        src4 = src[sl].reshape(NS, nch, BS)
        if nchp != nch:
            src4 = jnp.pad(src4, ((0, 0), (0, nchp - nch), (0, 0)))
        oma, omb, cnt2 = _sc_scatter(mma, mmb, src4, zeros)
        oms.append((oma, omb))
        csum = cnt2[0, :, 0] + cnt2[1, :, 0]
        cnt = csum if cnt is None else cnt + csum

    rec = 1.0 / jnp.clip(cnt, 1.0)
    rec128 = jnp.broadcast_to(rec[:, None], (N_NODES, 128))

    h = _node_mlp(oms[0][0], oms[1][0], oms[0][1], oms[1][1], rec128, x_s,
                  x_u, U1, c1.reshape(1, 1280), U2, c2.reshape(1, 128))
    return _batchnorm(h, gamma, beta)


# speedup vs baseline: 1.0114x; 1.0114x over previous
import functools

import jax
import jax.numpy as jnp
from jax import lax
from jax.experimental import pallas as pl
from jax.experimental.pallas import tpu as pltpu
from jax.experimental.pallas import tpu_sc as plsc

SLOPE = 0.2
E_TILE = 2560
N_TILE = 1000

N_NODES = 10000
N_EDGES = 320000
NS = 16
E_CHUNKS = (102400, 217600)
E_OFFS = (0, 102400)
BS = 80
SBLK = 64
NW = 32
G = 40
NPT = 624
NREM = N_NODES - NS * NPT
NPT = 624
NREM = N_NODES - NS * NPT


def _leaky(x):
    return jnp.where(x >= 0, x, SLOPE * x)


def _edge_mlp_body(xt_ref, ea_ref, w1a_ref, w1b_ref, b1_ref, w2_ref, b2_ref,
                   mma_ref, mmb_ref):
    h = xt_ref[...] @ w1a_ref[...] + ea_ref[...] @ w1b_ref[...] + b1_ref[...]
    h = _leaky(h)
    m = h @ w2_ref[...] + b2_ref[...]
    m2 = m * m
    m3 = m2 * m
    m4 = m2 * m2
    mma_ref[0] = m[:, :128]
    mma_ref[1] = m2[:, :128]
    mma_ref[2] = m3[:, :128]
    mma_ref[3] = m4[:, :128]
    mmb_ref[0] = m[:, 128:]
    mmb_ref[1] = m2[:, 128:]
    mmb_ref[2] = m3[:, 128:]
    mmb_ref[3] = m4[:, 128:]


def _edge_mlp(xt_g, ea_full, p, W1a, W1b, b1, W2, b2):
    e = xt_g.shape[0]
    grid = e // E_TILE
    off = E_OFFS[p] // E_TILE
    row_spec = pl.BlockSpec((E_TILE, 128), lambda i: (i, 0))
    ea_spec = pl.BlockSpec((E_TILE, 128), lambda i: (i + off, 0))
    full = lambda shape: pl.BlockSpec(shape, lambda i: (0,) * len(shape))
    out_sd = jax.ShapeDtypeStruct((4, e, 128), jnp.float32)
    return pl.pallas_call(
        _edge_mlp_body,
        grid=(grid,),
        in_specs=[row_spec, ea_spec,
                  full((128, 256)), full((128, 256)), full((1, 256)),
                  full((256, 256)), full((1, 256))],
        out_specs=[pl.BlockSpec((4, E_TILE, 128), lambda i: (0, i, 0))] * 2,
        out_shape=[out_sd] * 2,
    )(xt_g, ea_full, W1a, W1b, b1, W2, b2)


def _zero_slice(src_zeros, dst, s):
    row0 = pl.multiple_of(s * NPT, 8)
    pltpu.sync_copy(src_zeros.at[pl.ds(row0, NPT)], dst.at[pl.ds(row0, NPT)])

    @pl.when(s == NS - 1)
    def _():
        pltpu.sync_copy(src_zeros.at[pl.ds(NS * NPT, NREM)],
                        dst.at[pl.ds(NS * NPT, NREM)])


def _flush_slice(src_acc, dst, s):
    row0 = pl.multiple_of(s * NPT, 8)
    pltpu.sync_copy(src_acc.at[pl.ds(row0, NPT)], dst.at[pl.ds(row0, NPT)])

    @pl.when(s == NS - 1)
    def _():
        pltpu.sync_copy(src_acc.at[pl.ds(NS * NPT, NREM)],
                        dst.at[pl.ds(NS * NPT, NREM)])


def _make_sc_body(ept, blocks):
    def _sc_body(mma_ref, mmb_ref, src_ref, zer_ref,
                 out_a_ref, out_b_ref, outc_ref,
                 src_v, buf0, buf1, buf2, acc, g0, g1, g2, ss):
        c = lax.axis_index("c")
        s = lax.axis_index("s")
        e_base = pl.multiple_of(s * ept, 8)
        bufs = (buf0, buf1, buf2)
        gsems = (g0, g1, g2)

        def acc_at(t):
            return acc.at[src_v.at[t]]

        def stage_src(blk):
            pltpu.sync_copy(src_ref.at[s, pl.ds(blk * SBLK, SBLK)], src_v)

        def scatter_block(mm_ref, m, blk, live):
            t_base = blk * SBLK
            ntri = live // 3
            tail = live % 3

            def triple(i, carry):
                r0 = i * 3
                ds_ = [
                    pltpu.async_copy(
                        mm_ref.at[m,
                                  pl.ds(e_base + (t_base + r0 + k) * BS, BS)],
                        bufs[k], gsems[k])
                    for k in range(3)
                ]
                ss_ = []
                for k in range(3):
                    ds_[k].wait()
                    ss_.append(pltpu.async_copy(bufs[k], acc_at(r0 + k), ss,
                                                add=True))
                for d in ss_:
                    d.wait()
                return carry

            lax.fori_loop(0, ntri, triple, 0)
            for k in range(tail):
                r = ntri * 3 + k
                pltpu.sync_copy(
                    mm_ref.at[m, pl.ds(e_base + (t_base + r) * BS, BS)], buf0)
                st = pltpu.async_copy(buf0, acc_at(r), ss, add=True)
                st.wait()

        for j in range(2):
            m = c * 2 + j
            for half in range(2):
                mm_ref = mma_ref if half == 0 else mmb_ref
                out_ref = out_a_ref if half == 0 else out_b_ref
                _zero_slice(zer_ref, acc, s)
                plsc.subcore_barrier()
                for blk, live in enumerate(blocks):
                    stage_src(blk)
                    scatter_block(mm_ref, m, blk, live)
                plsc.subcore_barrier()
                _flush_slice(acc, out_ref.at[m], s)

        def fill(r, carry):
            for q in range(8):
                buf1[r, pl.ds(q * 16, 16)] = jnp.ones((16,), jnp.float32)
            return carry

        lax.fori_loop(0, BS, fill, 0)
        _zero_slice(zer_ref, acc, s)
        plsc.subcore_barrier()

        def count_block(nch):
            def cbody(i, carry):
                t0 = i * 2
                s0 = pltpu.async_copy(buf1, acc_at(t0), ss, add=True)
                s1 = pltpu.async_copy(buf1, acc_at(t0 + 1), ss, add=True)
                s0.wait()
                s1.wait()
                return carry

            lax.fori_loop(0, nch // 2, cbody, 0)
            if nch % 2:
                st = pltpu.async_copy(buf1, acc_at(nch - 1), ss, add=True)
                st.wait()

        for blk, live in enumerate(blocks):
            @pl.when(c == blk % 2)
            def _():
                stage_src(blk)
                count_block(live)

        plsc.subcore_barrier()

        @pl.when(c == 0)
        def _():
            _flush_slice(acc, outc_ref.at[0], s)

        @pl.when(c == 1)
        def _():
            _flush_slice(acc, outc_ref.at[1], s)

    return _sc_body


def _sc_scatter(mma, mmb, src4, zeros):
    e = mma.shape[1]
    ept = e // NS
    nch = ept // BS
    nblk = (nch + SBLK - 1) // SBLK
    blocks = [min(SBLK, nch - b * SBLK) for b in range(nblk)]
    f = pl.kernel(
        _make_sc_body(ept, blocks),
        out_type=[
            jax.ShapeDtypeStruct((4, N_NODES, 128), jnp.float32),
            jax.ShapeDtypeStruct((4, N_NODES, 128), jnp.float32),
            jax.ShapeDtypeStruct((2, N_NODES, 128), jnp.float32),
        ],
        mesh=plsc.VectorSubcoreMesh(core_axis_name="c", subcore_axis_name="s"),
        scratch_types=[
            pltpu.VMEM((SBLK, BS), jnp.int32),
            pltpu.VMEM((BS, 128), jnp.float32),
            pltpu.VMEM((BS, 128), jnp.float32),
            pltpu.VMEM((BS, 128), jnp.float32),
            pltpu.VMEM_SHARED((N_NODES, 128), jnp.float32),
            pltpu.SemaphoreType.DMA,
            pltpu.SemaphoreType.DMA,
            pltpu.SemaphoreType.DMA,
            pltpu.SemaphoreType.DMA,
        ],
    )
    return f(mma, mmb, src4, zeros)


def _make_gather_body(rpt, ncg):
    def _gather_body(xt_ref, idx_ref, out_ref, idx_v, gb0, gb1, gb2, gb3,
                     s0, s1, s2, s3, os):
        c = lax.axis_index("c")
        s = lax.axis_index("s")
        w = s * 2 + c
        base = pl.multiple_of(w * rpt, 8)
        pltpu.sync_copy(idx_ref.at[w], idx_v)
        gbs = (gb0, gb1, gb2, gb3)
        gsems = (s0, s1, s2, s3)

        def quad(i, carry):
            t0 = i * 4
            ds_ = [
                pltpu.async_copy(xt_ref.at[idx_v.at[t0 + k]], gbs[k],
                                 gsems[k])
                for k in range(4)
            ]
            os_ = []
            for k in range(4):
                ds_[k].wait()
                os_.append(pltpu.async_copy(
                    gbs[k], out_ref.at[pl.ds(base + (t0 + k) * G, G)], os))
            for d in os_:
                d.wait()
            return carry

        lax.fori_loop(0, ncg // 4, quad, 0)
        for k in range(ncg % 4):
            t = (ncg // 4) * 4 + k
            d = pltpu.async_copy(xt_ref.at[idx_v.at[t]], gb0, s0)
            d.wait()
            pltpu.sync_copy(gb0, out_ref.at[pl.ds(base + t * G, G)])

    return _gather_body


def _sc_gather(x_t, idx3):
    nw, ncg, _ = idx3.shape
    rpt = ncg * G
    f = pl.kernel(
        _make_gather_body(rpt, ncg),
        out_type=jax.ShapeDtypeStruct((nw * rpt, 128), jnp.float32),
        mesh=plsc.VectorSubcoreMesh(core_axis_name="c", subcore_axis_name="s"),
        scratch_types=[
            pltpu.VMEM((ncg, G), jnp.int32),
            pltpu.VMEM((G, 128), jnp.float32),
            pltpu.VMEM((G, 128), jnp.float32),
            pltpu.VMEM((G, 128), jnp.float32),
            pltpu.VMEM((G, 128), jnp.float32),
            pltpu.SemaphoreType.DMA,
            pltpu.SemaphoreType.DMA,
            pltpu.SemaphoreType.DMA,
            pltpu.SemaphoreType.DMA,
            pltpu.SemaphoreType.DMA,
        ],
    )
    return f(x_t, idx3)


def _node_body(oma0_ref, oma1_ref, omb0_ref, omb1_ref, rec_ref, xs_ref,
               xu_ref, u1_ref, c1_ref, u2_ref, c2_ref, h_ref):
    r = rec_ref[:, 0:1]

    def stats(om):
        mu1 = om[0] * r
        mu2 = om[1] * r
        mu3 = om[2] * r
        mu4 = om[3] * r
        var = _leaky(mu2 - mu1 * mu1)
        std = jnp.sqrt(var + 1e-6)
        cen3 = mu3 - 3.0 * mu1 * mu2 + 2.0 * mu1 * mu1 * mu1
        cen4 = (mu4 - 4.0 * mu1 * mu3 + 6.0 * mu1 * mu1 * mu2
                - 3.0 * mu1 * mu1 * mu1 * mu1)
        s3 = std * std * std
        return mu1, std, cen3 / s3, cen4 / (s3 * std)

    mu1a, stda, skewa, kurta = stats(oma0_ref[...] + oma1_ref[...])
    mu1b, stdb, skewb, kurtb = stats(omb0_ref[...] + omb1_ref[...])
    xu = jnp.broadcast_to(xu_ref[...], (N_TILE, 128))
    hin = jnp.concatenate([xs_ref[...], mu1a, mu1b, stda, stdb,
                           skewa, skewb, kurta, kurtb, xu], axis=1)
    z = _leaky(hin @ u1_ref[...] + c1_ref[...])
    h_ref[...] = z @ u2_ref[...] + c2_ref[...]


def _node_mlp(oma0, oma1, omb0, omb1, rec128, x_s, x_u, U1, c1, U2, c2):
    n = x_s.shape[0]
    grid = n // N_TILE
    full = lambda shape: pl.BlockSpec(shape, lambda i: (0,) * len(shape))
    om_spec = pl.BlockSpec((4, N_TILE, 128), lambda i: (0, i, 0))
    return pl.pallas_call(
        _node_body,
        grid=(grid,),
        in_specs=[om_spec, om_spec, om_spec, om_spec,
                  pl.BlockSpec((N_TILE, 128), lambda i: (i, 0)),
                  pl.BlockSpec((N_TILE, 128), lambda i: (i, 0)),
                  full((1, 128)),
                  full((1280, 1280)), full((1, 1280)),
                  full((1280, 128)), full((1, 128))],
        out_specs=pl.BlockSpec((N_TILE, 128), lambda i: (i, 0)),
        out_shape=jax.ShapeDtypeStruct((n, 128), jnp.float32),
    )(oma0, oma1, omb0, omb1, rec128, x_s, x_u, U1, c1, U2, c2)


def _bn_body(h_ref, g_ref, b_ref, out_ref):
    h = h_ref[...]
    mu = jnp.mean(h, axis=0, keepdims=True)
    v = jnp.mean((h - mu) ** 2, axis=0, keepdims=True)
    out_ref[...] = g_ref[...] * (h - mu) / jnp.sqrt(v + 1e-5) + b_ref[...]


def _batchnorm(h, gamma, beta):
    n = h.shape[0]
    return pl.pallas_call(
        _bn_body,
        in_specs=[pl.BlockSpec((n, 128), lambda: (0, 0)),
                  pl.BlockSpec((1, 128), lambda: (0, 0)),
                  pl.BlockSpec((1, 128), lambda: (0, 0))],
        out_specs=pl.BlockSpec((n, 128), lambda: (0, 0)),
        out_shape=jax.ShapeDtypeStruct((n, 128), jnp.float32),
    )(h, gamma.reshape(1, 128), beta.reshape(1, 128))


def kernel(x_s, x_t, edge_index, edge_attr, x_u, W1, b1, W2, b2, U1, c1, U2,
           c2, gamma, beta):
    src = edge_index[0]
    tgt = edge_index[1]

    W1a = W1[:128]
    W1b = W1[128:]

    zeros = jnp.zeros((N_NODES, 128), jnp.float32)
    b1r = b1.reshape(1, 256)
    b2r = b2.reshape(1, 256)

    oms = []
    cnt = None
    for p in range(len(E_CHUNKS)):
        ec = E_CHUNKS[p]
        sl = slice(E_OFFS[p], E_OFFS[p] + ec)
        xt_g = _sc_gather(x_t, tgt[sl].reshape(NW, ec // (NW * G), G))
        mma, mmb = _edge_mlp(xt_g, edge_attr, p, W1a, W1b, b1r, W2, b2r)
        nch = ec // (NS * BS)
        nchp = ((nch + SBLK - 1) // SBLK) * SBLK
        src4 = src[sl].reshape(NS, nch, BS)
        if nchp != nch:
            src4 = jnp.pad(src4, ((0, 0), (0, nchp - nch), (0, 0)))
        oma, omb, cnt2 = _sc_scatter(mma, mmb, src4, zeros)
        oms.append((oma, omb))
        csum = cnt2[0, :, 0] + cnt2[1, :, 0]
        cnt = csum if cnt is None else cnt + csum

    rec = 1.0 / jnp.clip(cnt, 1.0)
    rec128 = jnp.broadcast_to(rec[:, None], (N_NODES, 128))

    h = _node_mlp(oms[0][0], oms[1][0], oms[0][1], oms[1][1], rec128, x_s,
                  x_u, U1, c1.reshape(1, 1280), U2, c2.reshape(1, 128))
    return _batchnorm(h, gamma, beta)


# speedup vs baseline: 1.0586x; 1.0466x over previous
import functools

import jax
import jax.numpy as jnp
from jax import lax
from jax.experimental import pallas as pl
from jax.experimental.pallas import tpu as pltpu
from jax.experimental.pallas import tpu_sc as plsc

SLOPE = 0.2
E_TILE = 2000
N_TILE = 1000

N_NODES = 10000
N_EDGES = 320000
NS = 16
E_CHUNKS = (160000, 160000)
E_OFFS = (0, 160000)
BS = 80
SBLK = 64
NW = 32
G = 40
NPT = 624
NREM = N_NODES - NS * NPT
NPT = 624
NREM = N_NODES - NS * NPT


def _leaky(x):
    return jnp.where(x >= 0, x, SLOPE * x)


def _edge_mlp_body(xt_ref, ea_ref, w1a_ref, w1b_ref, b1_ref, w2_ref, b2_ref,
                   mma_ref, mmb_ref):
    h = xt_ref[...] @ w1a_ref[...] + ea_ref[...] @ w1b_ref[...] + b1_ref[...]
    h = _leaky(h)
    m = h @ w2_ref[...] + b2_ref[...]
    m2 = m * m
    m3 = m2 * m
    m4 = m2 * m2
    mma_ref[0] = m[:, :128]
    mma_ref[1] = m2[:, :128]
    mma_ref[2] = m3[:, :128]
    mma_ref[3] = m4[:, :128]
    mmb_ref[0] = m[:, 128:]
    mmb_ref[1] = m2[:, 128:]
    mmb_ref[2] = m3[:, 128:]
    mmb_ref[3] = m4[:, 128:]


def _edge_mlp(xt_g, ea_full, p, W1a, W1b, b1, W2, b2):
    e = xt_g.shape[0]
    grid = e // E_TILE
    off = E_OFFS[p] // E_TILE
    row_spec = pl.BlockSpec((E_TILE, 128), lambda i: (i, 0))
    ea_spec = pl.BlockSpec((E_TILE, 128), lambda i: (i + off, 0))
    full = lambda shape: pl.BlockSpec(shape, lambda i: (0,) * len(shape))
    out_sd = jax.ShapeDtypeStruct((4, e, 128), jnp.float32)
    return pl.pallas_call(
        _edge_mlp_body,
        grid=(grid,),
        in_specs=[row_spec, ea_spec,
                  full((128, 256)), full((128, 256)), full((1, 256)),
                  full((256, 256)), full((1, 256))],
        out_specs=[pl.BlockSpec((4, E_TILE, 128), lambda i: (0, i, 0))] * 2,
        out_shape=[out_sd] * 2,
    )(xt_g, ea_full, W1a, W1b, b1, W2, b2)


def _zero_slice(src_zeros, dst, s):
    row0 = pl.multiple_of(s * NPT, 8)
    pltpu.sync_copy(src_zeros.at[pl.ds(row0, NPT)], dst.at[pl.ds(row0, NPT)])

    @pl.when(s == NS - 1)
    def _():
        pltpu.sync_copy(src_zeros.at[pl.ds(NS * NPT, NREM)],
                        dst.at[pl.ds(NS * NPT, NREM)])


def _flush_slice(src_acc, dst, s):
    row0 = pl.multiple_of(s * NPT, 8)
    pltpu.sync_copy(src_acc.at[pl.ds(row0, NPT)], dst.at[pl.ds(row0, NPT)])

    @pl.when(s == NS - 1)
    def _():
        pltpu.sync_copy(src_acc.at[pl.ds(NS * NPT, NREM)],
                        dst.at[pl.ds(NS * NPT, NREM)])


def _make_sc_body(ept, blocks):
    def _sc_body(mma_ref, mmb_ref, src_ref, zer_ref,
                 out_a_ref, out_b_ref, outc_ref,
                 src_v, buf0, buf1, buf2, acc, g0, g1, g2, ss):
        c = lax.axis_index("c")
        s = lax.axis_index("s")
        e_base = pl.multiple_of(s * ept, 8)
        bufs = (buf0, buf1, buf2)
        gsems = (g0, g1, g2)

        def acc_at(t):
            return acc.at[src_v.at[t]]

        def stage_src(blk):
            pltpu.sync_copy(src_ref.at[s, pl.ds(blk * SBLK, SBLK)], src_v)

        def scatter_block(mm_ref, m, blk, live):
            t_base = blk * SBLK
            ntri = live // 3
            tail = live % 3

            def triple(i, carry):
                r0 = i * 3
                ds_ = [
                    pltpu.async_copy(
                        mm_ref.at[m,
                                  pl.ds(e_base + (t_base + r0 + k) * BS, BS)],
                        bufs[k], gsems[k])
                    for k in range(3)
                ]
                ss_ = []
                for k in range(3):
                    ds_[k].wait()
                    ss_.append(pltpu.async_copy(bufs[k], acc_at(r0 + k), ss,
                                                add=True))
                for d in ss_:
                    d.wait()
                return carry

            lax.fori_loop(0, ntri, triple, 0)
            for k in range(tail):
                r = ntri * 3 + k
                pltpu.sync_copy(
                    mm_ref.at[m, pl.ds(e_base + (t_base + r) * BS, BS)], buf0)
                st = pltpu.async_copy(buf0, acc_at(r), ss, add=True)
                st.wait()

        for j in range(2):
            m = c * 2 + j
            for half in range(2):
                mm_ref = mma_ref if half == 0 else mmb_ref
                out_ref = out_a_ref if half == 0 else out_b_ref
                _zero_slice(zer_ref, acc, s)
                plsc.subcore_barrier()
                for blk, live in enumerate(blocks):
                    stage_src(blk)
                    scatter_block(mm_ref, m, blk, live)
                plsc.subcore_barrier()
                _flush_slice(acc, out_ref.at[m], s)

        def fill(r, carry):
            for q in range(8):
                buf1[r, pl.ds(q * 16, 16)] = jnp.ones((16,), jnp.float32)
            return carry

        lax.fori_loop(0, BS, fill, 0)
        _zero_slice(zer_ref, acc, s)
        plsc.subcore_barrier()

        def count_block(nch):
            def cbody(i, carry):
                t0 = i * 2
                s0 = pltpu.async_copy(buf1, acc_at(t0), ss, add=True)
                s1 = pltpu.async_copy(buf1, acc_at(t0 + 1), ss, add=True)
                s0.wait()
                s1.wait()
                return carry

            lax.fori_loop(0, nch // 2, cbody, 0)
            if nch % 2:
                st = pltpu.async_copy(buf1, acc_at(nch - 1), ss, add=True)
                st.wait()

        for blk, live in enumerate(blocks):
            @pl.when(c == blk % 2)
            def _():
                stage_src(blk)
                count_block(live)

        plsc.subcore_barrier()

        @pl.when(c == 0)
        def _():
            _flush_slice(acc, outc_ref.at[0], s)

        @pl.when(c == 1)
        def _():
            _flush_slice(acc, outc_ref.at[1], s)

    return _sc_body


def _sc_scatter(mma, mmb, src4, zeros):
    e = mma.shape[1]
    ept = e // NS
    nch = ept // BS
    nblk = (nch + SBLK - 1) // SBLK
    blocks = [min(SBLK, nch - b * SBLK) for b in range(nblk)]
    f = pl.kernel(
        _make_sc_body(ept, blocks),
        out_type=[
            jax.ShapeDtypeStruct((4, N_NODES, 128), jnp.float32),
            jax.ShapeDtypeStruct((4, N_NODES, 128), jnp.float32),
            jax.ShapeDtypeStruct((2, N_NODES, 128), jnp.float32),
        ],
        mesh=plsc.VectorSubcoreMesh(core_axis_name="c", subcore_axis_name="s"),
        scratch_types=[
            pltpu.VMEM((SBLK, BS), jnp.int32),
            pltpu.VMEM((BS, 128), jnp.float32),
            pltpu.VMEM((BS, 128), jnp.float32),
            pltpu.VMEM((BS, 128), jnp.float32),
            pltpu.VMEM_SHARED((N_NODES, 128), jnp.float32),
            pltpu.SemaphoreType.DMA,
            pltpu.SemaphoreType.DMA,
            pltpu.SemaphoreType.DMA,
            pltpu.SemaphoreType.DMA,
        ],
    )
    return f(mma, mmb, src4, zeros)


def _make_gather_body(rpt, ncg):
    def _gather_body(xt_ref, idx_ref, out_ref, idx_v, gb0, gb1, gb2, gb3,
                     s0, s1, s2, s3, os):
        c = lax.axis_index("c")
        s = lax.axis_index("s")
        w = s * 2 + c
        base = pl.multiple_of(w * rpt, 8)
        pltpu.sync_copy(idx_ref.at[w], idx_v)
        gbs = (gb0, gb1, gb2, gb3)
        gsems = (s0, s1, s2, s3)

        def quad(i, carry):
            t0 = i * 4
            ds_ = [
                pltpu.async_copy(xt_ref.at[idx_v.at[t0 + k]], gbs[k],
                                 gsems[k])
                for k in range(4)
            ]
            os_ = []
            for k in range(4):
                ds_[k].wait()
                os_.append(pltpu.async_copy(
                    gbs[k], out_ref.at[pl.ds(base + (t0 + k) * G, G)], os))
            for d in os_:
                d.wait()
            return carry

        lax.fori_loop(0, ncg // 4, quad, 0)
        for k in range(ncg % 4):
            t = (ncg // 4) * 4 + k
            d = pltpu.async_copy(xt_ref.at[idx_v.at[t]], gb0, s0)
            d.wait()
            pltpu.sync_copy(gb0, out_ref.at[pl.ds(base + t * G, G)])

    return _gather_body


def _sc_gather(x_t, idx3):
    nw, ncg, _ = idx3.shape
    rpt = ncg * G
    f = pl.kernel(
        _make_gather_body(rpt, ncg),
        out_type=jax.ShapeDtypeStruct((nw * rpt, 128), jnp.float32),
        mesh=plsc.VectorSubcoreMesh(core_axis_name="c", subcore_axis_name="s"),
        scratch_types=[
            pltpu.VMEM((ncg, G), jnp.int32),
            pltpu.VMEM((G, 128), jnp.float32),
            pltpu.VMEM((G, 128), jnp.float32),
            pltpu.VMEM((G, 128), jnp.float32),
            pltpu.VMEM((G, 128), jnp.float32),
            pltpu.SemaphoreType.DMA,
            pltpu.SemaphoreType.DMA,
            pltpu.SemaphoreType.DMA,
            pltpu.SemaphoreType.DMA,
            pltpu.SemaphoreType.DMA,
        ],
    )
    return f(x_t, idx3)


def _node_body(oma0_ref, oma1_ref, omb0_ref, omb1_ref, rec_ref, xs_ref,
               xu_ref, u1_ref, c1_ref, u2_ref, c2_ref, h_ref):
    r = rec_ref[:, 0:1]

    def stats(om):
        mu1 = om[0] * r
        mu2 = om[1] * r
        mu3 = om[2] * r
        mu4 = om[3] * r
        var = _leaky(mu2 - mu1 * mu1)
        std = jnp.sqrt(var + 1e-6)
        cen3 = mu3 - 3.0 * mu1 * mu2 + 2.0 * mu1 * mu1 * mu1
        cen4 = (mu4 - 4.0 * mu1 * mu3 + 6.0 * mu1 * mu1 * mu2
                - 3.0 * mu1 * mu1 * mu1 * mu1)
        s3 = std * std * std
        return mu1, std, cen3 / s3, cen4 / (s3 * std)

    mu1a, stda, skewa, kurta = stats(oma0_ref[...] + oma1_ref[...])
    mu1b, stdb, skewb, kurtb = stats(omb0_ref[...] + omb1_ref[...])
    xu = jnp.broadcast_to(xu_ref[...], (N_TILE, 128))
    hin = jnp.concatenate([xs_ref[...], mu1a, mu1b, stda, stdb,
                           skewa, skewb, kurta, kurtb, xu], axis=1)
    z = _leaky(hin @ u1_ref[...] + c1_ref[...])
    h_ref[...] = z @ u2_ref[...] + c2_ref[...]


def _node_mlp(oma0, oma1, omb0, omb1, rec128, x_s, x_u, U1, c1, U2, c2):
    n = x_s.shape[0]
    grid = n // N_TILE
    full = lambda shape: pl.BlockSpec(shape, lambda i: (0,) * len(shape))
    om_spec = pl.BlockSpec((4, N_TILE, 128), lambda i: (0, i, 0))
    return pl.pallas_call(
        _node_body,
        grid=(grid,),
        in_specs=[om_spec, om_spec, om_spec, om_spec,
                  pl.BlockSpec((N_TILE, 128), lambda i: (i, 0)),
                  pl.BlockSpec((N_TILE, 128), lambda i: (i, 0)),
                  full((1, 128)),
                  full((1280, 1280)), full((1, 1280)),
                  full((1280, 128)), full((1, 128))],
        out_specs=pl.BlockSpec((N_TILE, 128), lambda i: (i, 0)),
        out_shape=jax.ShapeDtypeStruct((n, 128), jnp.float32),
    )(oma0, oma1, omb0, omb1, rec128, x_s, x_u, U1, c1, U2, c2)


def _bn_body(h_ref, g_ref, b_ref, out_ref):
    h = h_ref[...]
    mu = jnp.mean(h, axis=0, keepdims=True)
    v = jnp.mean((h - mu) ** 2, axis=0, keepdims=True)
    out_ref[...] = g_ref[...] * (h - mu) / jnp.sqrt(v + 1e-5) + b_ref[...]


def _batchnorm(h, gamma, beta):
    n = h.shape[0]
    return pl.pallas_call(
        _bn_body,
        in_specs=[pl.BlockSpec((n, 128), lambda: (0, 0)),
                  pl.BlockSpec((1, 128), lambda: (0, 0)),
                  pl.BlockSpec((1, 128), lambda: (0, 0))],
        out_specs=pl.BlockSpec((n, 128), lambda: (0, 0)),
        out_shape=jax.ShapeDtypeStruct((n, 128), jnp.float32),
    )(h, gamma.reshape(1, 128), beta.reshape(1, 128))


def kernel(x_s, x_t, edge_index, edge_attr, x_u, W1, b1, W2, b2, U1, c1, U2,
           c2, gamma, beta):
    src = edge_index[0]
    tgt = edge_index[1]

    W1a = W1[:128]
    W1b = W1[128:]

    zeros = jnp.zeros((N_NODES, 128), jnp.float32)
    b1r = b1.reshape(1, 256)
    b2r = b2.reshape(1, 256)

    oms = []
    cnt = None
    for p in range(len(E_CHUNKS)):
        ec = E_CHUNKS[p]
        sl = slice(E_OFFS[p], E_OFFS[p] + ec)
        xt_g = _sc_gather(x_t, tgt[sl].reshape(NW, ec // (NW * G), G))
        mma, mmb = _edge_mlp(xt_g, edge_attr, p, W1a, W1b, b1r, W2, b2r)
        nch = ec // (NS * BS)
        nchp = ((nch + SBLK - 1) // SBLK) * SBLK
        src4 = src[sl].reshape(NS, nch, BS)
        if nchp != nch:
            src4 = jnp.pad(src4, ((0, 0), (0, nchp - nch), (0, 0)))
        oma, omb, cnt2 = _sc_scatter(mma, mmb, src4, zeros)
        oms.append((oma, omb))
        csum = cnt2[0, :, 0] + cnt2[1, :, 0]
        cnt = csum if cnt is None else cnt + csum

    rec = 1.0 / jnp.clip(cnt, 1.0)
    rec128 = jnp.broadcast_to(rec[:, None], (N_NODES, 128))

    h = _node_mlp(oms[0][0], oms[1][0], oms[0][1], oms[1][1], rec128, x_s,
                  x_u, U1, c1.reshape(1, 1280), U2, c2.reshape(1, 128))
    return _batchnorm(h, gamma, beta)
